# Initial kernel scaffold; baseline (speedup 1.0000x reference)
import functools

import jax
import jax.numpy as jnp
from jax import lax
from jax.experimental import pallas as pl
from jax.experimental.pallas import tpu as pltpu
from jax.experimental.pallas import tpu_sc as plsc

NV = 10000
NE = 20000
NH = 128
NI = 320000

HC = NH // 2
NCORES = 2
NTILES = 16
LANES = 16
CHUNK = 80
PER_TILE = NI // NTILES


def _sc_phase(n_src, n_dst):
  n_chunks = PER_TILE // CHUNK
  dst_rows = n_dst // NTILES
  wseg = NI // NTILES

  mesh = plsc.VectorSubcoreMesh(
      core_axis_name="c", subcore_axis_name="s",
      num_cores=NCORES, num_subcores=NTILES)

  @functools.partial(
      pl.kernel,
      out_type=jax.ShapeDtypeStruct((n_dst, NH), jnp.float32),
      mesh=mesh,
      scratch_types=dict(
          acc=pltpu.VMEM_SHARED((n_dst, HC), jnp.float32),
          wtab=pltpu.VMEM_SHARED((NI,), jnp.float32),
          rtab=pltpu.VMEM_SHARED((n_dst,), jnp.float32),
          sidx=pltpu.VMEM((CHUNK,), jnp.int32),
          didx=pltpu.VMEM((CHUNK,), jnp.int32),
          woff=pltpu.VMEM((CHUNK,), jnp.int32),
          gidx=pltpu.VMEM((CHUNK,), jnp.int32),
          rows=pltpu.VMEM((CHUNK, HC), jnp.float32),
          wv=pltpu.VMEM((CHUNK,), jnp.float32),
          rv=pltpu.VMEM((CHUNK,), jnp.float32),
          sv=pltpu.VMEM((CHUNK,), jnp.float32),
      ),
  )
  def phase(src2, init, srcidx, dstidx, widx, wtab_hbm, rtab_hbm, out,
            acc, wtab, rtab, sidx, didx, woff, gidx, rows, wv, rv, sv):
    c = lax.axis_index("c")
    s = lax.axis_index("s")

    pltpu.sync_copy(wtab_hbm.at[pl.ds(s * wseg, wseg)],
                    wtab.at[pl.ds(s * wseg, wseg)])
    pltpu.sync_copy(rtab_hbm.at[pl.ds(s * dst_rows, dst_rows)],
                    rtab.at[pl.ds(s * dst_rows, dst_rows)])
    pltpu.sync_copy(init.at[pl.ds(s * dst_rows, dst_rows), pl.ds(c * HC, HC)],
                    acc.at[pl.ds(s * dst_rows, dst_rows)])
    plsc.subcore_barrier()

    base_t = s * PER_TILE

    def chunk_body(k, carry):
      base = base_t + k * CHUNK
      pltpu.sync_copy(srcidx.at[pl.ds(base, CHUNK)], sidx)
      pltpu.sync_copy(dstidx.at[pl.ds(base, CHUNK)], didx)
      pltpu.sync_copy(widx.at[pl.ds(base, CHUNK)], woff)
      for g in range(CHUNK // LANES):
        sl = pl.ds(g * LANES, LANES)
        gidx[sl] = sidx[sl] * 2 + c
      pltpu.sync_copy(src2.at[gidx], rows)
      pltpu.sync_copy(wtab.at[woff], wv)
      pltpu.sync_copy(rtab.at[didx], rv)
      for g in range(CHUNK // LANES):
        sl = pl.ds(g * LANES, LANES)
        sv[sl] = wv[sl] * rv[sl]
      for r in range(CHUNK):
        spl = plsc.load_gather(sv, [jnp.full((LANES,), r, jnp.int32)])
        for q in range(HC // LANES):
          qs = pl.ds(q * LANES, LANES)
          rows[r, qs] = rows[r, qs] * spl
      pltpu.sync_copy(rows, acc.at[didx], add=True)
      return carry

    lax.fori_loop(0, n_chunks, chunk_body, 0)

    plsc.subcore_barrier()
    pltpu.sync_copy(acc.at[pl.ds(s * dst_rows, dst_rows)],
                    out.at[pl.ds(s * dst_rows, dst_rows), pl.ds(c * HC, HC)])

  return phase


_sc_v2e = _sc_phase(NV, NE)
_sc_e2v = _sc_phase(NE, NV)


BN_V = 2000
BN_E = 2000


def _tc1_body(v_ref, vw_ref, vrs_ref, W_ref, b_ref,
              vew_ref, vpre_ref, rv_ref):
  x = v_ref[...]
  act = jnp.maximum(
      jnp.dot(x, W_ref[...], preferred_element_type=jnp.float32) + b_ref[...],
      0.0)
  vw = vw_ref[...]
  r = 1.0 / vrs_ref[...]
  vew_ref[...] = act * vw
  vpre_ref[...] = x * vw * r
  rv_ref[...] = r


def _tc1e_body(e_ref, ers_ref, epre_ref, re_ref):
  r = 1.0 / ers_ref[...]
  epre_ref[...] = e_ref[...] * r
  re_ref[...] = r


def _tc2_body(eo_ref, ew_ref, W_ref, b_ref, evw_ref):
  act = jnp.maximum(
      jnp.dot(eo_ref[...], W_ref[...], preferred_element_type=jnp.float32)
      + b_ref[...], 0.0)
  evw_ref[...] = act * ew_ref[...]


def _row_spec(bn, width):
  return pl.BlockSpec((bn, width), lambda i: (i, 0))


def _full_spec(shape):
  return pl.BlockSpec(shape, lambda i: (0, 0))


_tc1 = pl.pallas_call(
    _tc1_body,
    grid=(NV // BN_V,),
    in_specs=[_row_spec(BN_V, NH), _row_spec(BN_V, 1), _row_spec(BN_V, 1),
              _full_spec((NH, NH)), _full_spec((1, NH))],
    out_specs=[_row_spec(BN_V, NH), _row_spec(BN_V, NH), _row_spec(BN_V, 1)],
    out_shape=[jax.ShapeDtypeStruct((NV, NH), jnp.float32),
               jax.ShapeDtypeStruct((NV, NH), jnp.float32),
               jax.ShapeDtypeStruct((NV, 1), jnp.float32)],
)

_tc1e = pl.pallas_call(
    _tc1e_body,
    grid=(NE // BN_E,),
    in_specs=[_row_spec(BN_E, NH), _row_spec(BN_E, 1)],
    out_specs=[_row_spec(BN_E, NH), _row_spec(BN_E, 1)],
    out_shape=[jax.ShapeDtypeStruct((NE, NH), jnp.float32),
               jax.ShapeDtypeStruct((NE, 1), jnp.float32)],
)

_tc2 = pl.pallas_call(
    _tc2_body,
    grid=(NE // BN_E,),
    in_specs=[_row_spec(BN_E, NH), _row_spec(BN_E, 1),
              _full_spec((NH, NH)), _full_spec((1, NH))],
    out_specs=_row_spec(BN_E, NH),
    out_shape=jax.ShapeDtypeStruct((NE, NH), jnp.float32),
)


@jax.jit
def kernel(v, e, player_idx, game_idx, idx, W_v2e, W_e2v, b_v, b_e,
           v_weight, e_weight, v_reg_weight, e_reg_weight,
           v_reg_sum, e_reg_sum):
  ve_w, v_pre, rv = _tc1(v, v_weight, v_reg_sum, W_v2e,
                         b_v.reshape(1, NH))
  e_pre, re = _tc1e(e, e_reg_sum)

  e_out = _sc_v2e(ve_w.reshape(2 * NV, HC), e_pre, player_idx, game_idx,
                  idx, v_reg_weight.reshape(NI), re.reshape(NE))

  evw = _tc2(e_out, e_weight, W_e2v, b_e.reshape(1, NH))

  v_out = _sc_e2v(evw.reshape(2 * NE, HC), v_pre, game_idx, player_idx,
                  idx, e_reg_weight.reshape(NI), rv.reshape(NV))

  return (v_out, e_out)


# speedup vs baseline: 2.1843x; 2.1843x over previous
import functools

import jax
import jax.numpy as jnp
from jax import lax
from jax.experimental import pallas as pl
from jax.experimental.pallas import tpu as pltpu
from jax.experimental.pallas import tpu_sc as plsc

NV = 10000
NE = 20000
NH = 128
NI = 320000

HC = NH // 2
NCORES = 2
NTILES = 16
LANES = 16
CHUNK = 80
PER_TILE = NI // NTILES

def _sc_phase(n_src, n_dst):
  n_chunks = PER_TILE // CHUNK
  dst_rows = n_dst // NTILES
  wseg = NI // NTILES

  mesh = plsc.VectorSubcoreMesh(
      core_axis_name="c", subcore_axis_name="s",
      num_cores=NCORES, num_subcores=NTILES)

  @functools.partial(
      pl.kernel,
      out_type=jax.ShapeDtypeStruct((n_dst, NH), jnp.float32),
      mesh=mesh,
      compiler_params=pltpu.CompilerParams(use_tc_tiling_on_sc=False,
                                           needs_layout_passes=False),
      scratch_types=dict(
          acc=pltpu.VMEM_SHARED((n_dst, HC), jnp.float32),
          wtab=pltpu.VMEM_SHARED((NI,), jnp.float32),
          rtab=pltpu.VMEM_SHARED((n_dst,), jnp.float32),
          sidx=pltpu.VMEM((CHUNK,), jnp.int32),
          didx=pltpu.VMEM((CHUNK,), jnp.int32),
          woff=pltpu.VMEM((CHUNK,), jnp.int32),
          gidx=pltpu.VMEM((CHUNK,), jnp.int32),
          rows=pltpu.VMEM((CHUNK, HC), jnp.float32),
          wv=pltpu.VMEM((CHUNK,), jnp.float32),
          rv=pltpu.VMEM((CHUNK,), jnp.float32),
      ),
  )
  def phase(src2, init, srcidx, dstidx, widx, wtab_hbm, rtab_hbm, out,
            acc, wtab, rtab, sidx, didx, woff, gidx, rows, wv, rv):
    c = lax.axis_index("c")
    s = lax.axis_index("s")

    pltpu.sync_copy(wtab_hbm.at[pl.ds(s * wseg, wseg)],
                    wtab.at[pl.ds(s * wseg, wseg)])
    rseg = n_dst // 10
    @pl.when(s < 10)
    def _():
      pltpu.sync_copy(rtab_hbm.at[pl.ds(s * rseg, rseg)],
                      rtab.at[pl.ds(s * rseg, rseg)])
    pltpu.sync_copy(init.at[pl.ds(s * dst_rows, dst_rows), pl.ds(c * HC, HC)],
                    acc.at[pl.ds(s * dst_rows, dst_rows)])
    plsc.subcore_barrier()

    base_t = s * PER_TILE
    lane_iota = lax.iota(jnp.int32, LANES)

    def chunk_body(k, carry):
      base = base_t + k * CHUNK
      pltpu.sync_copy(srcidx.at[pl.ds(base, CHUNK)], sidx)
      pltpu.sync_copy(dstidx.at[pl.ds(base, CHUNK)], didx)
      pltpu.sync_copy(widx.at[pl.ds(base, CHUNK)], woff)
      for g in range(CHUNK // LANES):
        sl = pl.ds(g * LANES, LANES)
        gidx[sl] = sidx[sl] * 2 + c
      pltpu.sync_copy(src2.at[gidx], rows)
      pltpu.sync_copy(wtab.at[woff], wv)
      pltpu.sync_copy(rtab.at[didx], rv)
      for g in range(CHUNK // LANES):
        sl = pl.ds(g * LANES, LANES)
        s16 = wv[sl] * rv[sl]
        for i in range(LANES):
          r = g * LANES + i
          mask = (lane_iota == i).astype(jnp.float32)
          spl = jnp.full((LANES,), jnp.sum(s16 * mask))
          for q in range(HC // LANES):
            qs = pl.ds(q * LANES, LANES)
            rows[r, qs] = rows[r, qs] * spl
      pltpu.sync_copy(rows, acc.at[didx], add=True)
      return carry

    lax.fori_loop(0, n_chunks, chunk_body, 0)

    plsc.subcore_barrier()
    pltpu.sync_copy(acc.at[pl.ds(s * dst_rows, dst_rows)],
                    out.at[pl.ds(s * dst_rows, dst_rows), pl.ds(c * HC, HC)])

  return phase


_sc_v2e = _sc_phase(NV, NE)
_sc_e2v = _sc_phase(NE, NV)


BN_V = 2000
BN_E = 2000


def _tc1_body(v_ref, vw_ref, vrs_ref, W_ref, b_ref,
              vew_ref, vpre_ref, rv_ref):
  x = v_ref[...]
  act = jnp.maximum(
      jnp.dot(x, W_ref[...], preferred_element_type=jnp.float32) + b_ref[...],
      0.0)
  vw = vw_ref[...]
  r = 1.0 / vrs_ref[...]
  vew_ref[...] = act * vw
  vpre_ref[...] = x * vw * r
  rv_ref[...] = r


def _tc1e_body(e_ref, ers_ref, epre_ref, re_ref):
  r = 1.0 / ers_ref[...]
  epre_ref[...] = e_ref[...] * r
  re_ref[...] = r


def _tc2_body(eo_ref, ew_ref, W_ref, b_ref, evw_ref):
  act = jnp.maximum(
      jnp.dot(eo_ref[...], W_ref[...], preferred_element_type=jnp.float32)
      + b_ref[...], 0.0)
  evw_ref[...] = act * ew_ref[...]


def _row_spec(bn, width):
  return pl.BlockSpec((bn, width), lambda i: (i, 0))


def _full_spec(shape):
  return pl.BlockSpec(shape, lambda i: (0, 0))


_tc1 = pl.pallas_call(
    _tc1_body,
    grid=(NV // BN_V,),
    in_specs=[_row_spec(BN_V, NH), _row_spec(BN_V, 1), _row_spec(BN_V, 1),
              _full_spec((NH, NH)), _full_spec((1, NH))],
    out_specs=[_row_spec(BN_V, NH), _row_spec(BN_V, NH), _row_spec(BN_V, 1)],
    out_shape=[jax.ShapeDtypeStruct((NV, NH), jnp.float32),
               jax.ShapeDtypeStruct((NV, NH), jnp.float32),
               jax.ShapeDtypeStruct((NV, 1), jnp.float32)],
)

_tc1e = pl.pallas_call(
    _tc1e_body,
    grid=(NE // BN_E,),
    in_specs=[_row_spec(BN_E, NH), _row_spec(BN_E, 1)],
    out_specs=[_row_spec(BN_E, NH), _row_spec(BN_E, 1)],
    out_shape=[jax.ShapeDtypeStruct((NE, NH), jnp.float32),
               jax.ShapeDtypeStruct((NE, 1), jnp.float32)],
)

_tc2 = pl.pallas_call(
    _tc2_body,
    grid=(NE // BN_E,),
    in_specs=[_row_spec(BN_E, NH), _row_spec(BN_E, 1),
              _full_spec((NH, NH)), _full_spec((1, NH))],
    out_specs=_row_spec(BN_E, NH),
    out_shape=jax.ShapeDtypeStruct((NE, NH), jnp.float32),
)


@jax.jit
def kernel(v, e, player_idx, game_idx, idx, W_v2e, W_e2v, b_v, b_e,
           v_weight, e_weight, v_reg_weight, e_reg_weight,
           v_reg_sum, e_reg_sum):
  ve_w, v_pre, rv = _tc1(v, v_weight, v_reg_sum, W_v2e,
                         b_v.reshape(1, NH))
  e_pre, re = _tc1e(e, e_reg_sum)

  e_out = _sc_v2e(ve_w.reshape(2 * NV, HC), e_pre, player_idx, game_idx,
                  idx, v_reg_weight.reshape(NI), re.reshape(NE))

  evw = _tc2(e_out, e_weight, W_e2v, b_e.reshape(1, NH))

  v_out = _sc_e2v(evw.reshape(2 * NE, HC), v_pre, game_idx, player_idx,
                  idx, e_reg_weight.reshape(NI), rv.reshape(NV))

  return (v_out, e_out)


# speedup vs baseline: 2.3336x; 1.0684x over previous
import functools

import jax
import jax.numpy as jnp
from jax import lax
from jax.experimental import pallas as pl
from jax.experimental.pallas import tpu as pltpu
from jax.experimental.pallas import tpu_sc as plsc

NV = 10000
NE = 20000
NH = 128
NI = 320000

HC = NH // 2
NCORES = 2
NTILES = 16
LANES = 16
CHUNK = 80
PER_TILE = NI // NTILES


def _sc_phase(n_src, n_dst):
  n_chunks = PER_TILE // CHUNK
  dst_rows = n_dst // NTILES
  wseg = NI // NTILES

  mesh = plsc.VectorSubcoreMesh(
      core_axis_name="c", subcore_axis_name="s",
      num_cores=NCORES, num_subcores=NTILES)

  @functools.partial(
      pl.kernel,
      out_type=jax.ShapeDtypeStruct((n_dst, NH), jnp.float32),
      mesh=mesh,
      compiler_params=pltpu.CompilerParams(use_tc_tiling_on_sc=False,
                                           needs_layout_passes=False),
      scratch_types=dict(
          acc=pltpu.VMEM_SHARED((n_dst, HC), jnp.float32),
          wtab=pltpu.VMEM_SHARED((NI,), jnp.float32),
          sidx=pltpu.VMEM((CHUNK,), jnp.int32),
          didx=pltpu.VMEM((CHUNK,), jnp.int32),
          woff=pltpu.VMEM((CHUNK,), jnp.int32),
          gidx=pltpu.VMEM((CHUNK,), jnp.int32),
          rows=pltpu.VMEM((CHUNK, HC), jnp.float32),
          wv=pltpu.VMEM((CHUNK,), jnp.float32),
      ),
  )
  def phase(src2, init, srcidx, dstidx, widx, wtab_hbm, out,
            acc, wtab, sidx, didx, woff, gidx, rows, wv):
    c = lax.axis_index("c")
    s = lax.axis_index("s")

    pltpu.sync_copy(wtab_hbm.at[pl.ds(s * wseg, wseg)],
                    wtab.at[pl.ds(s * wseg, wseg)])
    pltpu.sync_copy(init.at[pl.ds(s * dst_rows, dst_rows), pl.ds(c * HC, HC)],
                    acc.at[pl.ds(s * dst_rows, dst_rows)])
    plsc.subcore_barrier()

    base_t = s * PER_TILE

    def chunk_body(k, carry):
      base = base_t + k * CHUNK
      pltpu.sync_copy(srcidx.at[pl.ds(base, CHUNK)], sidx)
      pltpu.sync_copy(dstidx.at[pl.ds(base, CHUNK)], didx)
      pltpu.sync_copy(widx.at[pl.ds(base, CHUNK)], woff)
      for g in range(CHUNK // LANES):
        sl = pl.ds(g * LANES, LANES)
        gidx[sl] = sidx[sl] * 2 + c
      pltpu.sync_copy(src2.at[gidx], rows)
      pltpu.sync_copy(wtab.at[woff], wv)
      for g in range(CHUNK // LANES):
        sl = pl.ds(g * LANES, LANES)
        s16 = wv[sl]
        for i in range(LANES):
          r = g * LANES + i
          spl = jnp.full((LANES,), s16[i])
          for q in range(HC // LANES):
            qs = pl.ds(q * LANES, LANES)
            rows[r, qs] = rows[r, qs] * spl
      pltpu.sync_copy(rows, acc.at[didx], add=True)
      return carry

    lax.fori_loop(0, n_chunks, chunk_body, 0)

    plsc.subcore_barrier()
    pltpu.sync_copy(acc.at[pl.ds(s * dst_rows, dst_rows)],
                    out.at[pl.ds(s * dst_rows, dst_rows), pl.ds(c * HC, HC)])

  return phase


_sc_v2e = _sc_phase(NV, NE)
_sc_e2v = _sc_phase(NE, NV)


BN_V = 2000
BN_E = 2000


def _tc1_body(v_ref, vw_ref, W_ref, b_ref, vew_ref, vpre_ref):
  x = v_ref[...]
  act = jnp.maximum(
      jnp.dot(x, W_ref[...], preferred_element_type=jnp.float32) + b_ref[...],
      0.0)
  vw = vw_ref[...]
  vew_ref[...] = act * vw
  vpre_ref[...] = x * vw


def _tc2_body(acc_ref, ew_ref, ers_ref, W_ref, b_ref, eout_ref, evw_ref):
  e_out = acc_ref[...] * (1.0 / ers_ref[...])
  act = jnp.maximum(
      jnp.dot(e_out, W_ref[...], preferred_element_type=jnp.float32)
      + b_ref[...], 0.0)
  eout_ref[...] = e_out
  evw_ref[...] = act * ew_ref[...]


def _tc3_body(acc_ref, vrs_ref, vout_ref):
  vout_ref[...] = acc_ref[...] * (1.0 / vrs_ref[...])


def _row_spec(bn, width):
  return pl.BlockSpec((bn, width), lambda i: (i, 0))


def _full_spec(shape):
  return pl.BlockSpec(shape, lambda i: (0, 0))


_tc1 = pl.pallas_call(
    _tc1_body,
    grid=(NV // BN_V,),
    in_specs=[_row_spec(BN_V, NH), _row_spec(BN_V, 1),
              _full_spec((NH, NH)), _full_spec((1, NH))],
    out_specs=[_row_spec(BN_V, NH), _row_spec(BN_V, NH)],
    out_shape=[jax.ShapeDtypeStruct((NV, NH), jnp.float32),
               jax.ShapeDtypeStruct((NV, NH), jnp.float32)],
)

_tc2 = pl.pallas_call(
    _tc2_body,
    grid=(NE // BN_E,),
    in_specs=[_row_spec(BN_E, NH), _row_spec(BN_E, 1), _row_spec(BN_E, 1),
              _full_spec((NH, NH)), _full_spec((1, NH))],
    out_specs=[_row_spec(BN_E, NH), _row_spec(BN_E, NH)],
    out_shape=[jax.ShapeDtypeStruct((NE, NH), jnp.float32),
               jax.ShapeDtypeStruct((NE, NH), jnp.float32)],
)

_tc3 = pl.pallas_call(
    _tc3_body,
    grid=(NV // BN_V,),
    in_specs=[_row_spec(BN_V, NH), _row_spec(BN_V, 1)],
    out_specs=_row_spec(BN_V, NH),
    out_shape=jax.ShapeDtypeStruct((NV, NH), jnp.float32),
)


@jax.jit
def kernel(v, e, player_idx, game_idx, idx, W_v2e, W_e2v, b_v, b_e,
           v_weight, e_weight, v_reg_weight, e_reg_weight,
           v_reg_sum, e_reg_sum):
  ve_w, v_pre = _tc1(v, v_weight, W_v2e, b_v.reshape(1, NH))

  acc_e = _sc_v2e(ve_w.reshape(2 * NV, HC), e, player_idx, game_idx,
                  idx, v_reg_weight.reshape(NI))

  e_out, evw = _tc2(acc_e, e_weight, e_reg_sum, W_e2v, b_e.reshape(1, NH))

  acc_v = _sc_e2v(evw.reshape(2 * NE, HC), v_pre, game_idx, player_idx,
                  idx, e_reg_weight.reshape(NI))

  v_out = _tc3(acc_v, v_reg_sum)

  return (v_out, e_out)


# speedup vs baseline: 3.6457x; 1.5623x over previous
import functools

import jax
import jax.numpy as jnp
from jax import lax
from jax.experimental import pallas as pl
from jax.experimental.pallas import tpu as pltpu
from jax.experimental.pallas import tpu_sc as plsc

NV = 10000
NE = 20000
NH = 128
NI = 320000

HC = NH // 2
NCORES = 2
NTILES = 16
LANES = 16
CHUNK = 80
PER_TILE = NI // NTILES
NCHUNKS = PER_TILE // CHUNK


def _sc_phase(n_src, n_dst):
  dst_rows = n_dst // NTILES

  mesh = plsc.VectorSubcoreMesh(
      core_axis_name="c", subcore_axis_name="s",
      num_cores=NCORES, num_subcores=NTILES)

  @functools.partial(
      pl.kernel,
      out_type=jax.ShapeDtypeStruct((n_dst, NH), jnp.float32),
      mesh=mesh,
      compiler_params=pltpu.CompilerParams(use_tc_tiling_on_sc=False,
                                           needs_layout_passes=False),
      scratch_types=dict(
          acc=pltpu.VMEM_SHARED((n_dst, HC), jnp.float32),
          pidx=pltpu.VMEM((NCHUNKS, CHUNK), jnp.int32),
          didx=pltpu.VMEM((NCHUNKS, CHUNK), jnp.int32),
          woff=pltpu.VMEM((CHUNK,), jnp.int32),
          rows=pltpu.VMEM((CHUNK, HC), jnp.float32),
          wv=pltpu.VMEM((CHUNK,), jnp.float32),
          sem_r=pltpu.SemaphoreType.DMA,
          sem_w=pltpu.SemaphoreType.DMA,
      ),
  )
  def phase(src3, init, pidx3, didx3, widx3, wtab_hbm, out,
            acc, pidx, didx, woff, rows, wv, sem_r, sem_w):
    c = lax.axis_index("c")
    s = lax.axis_index("s")

    pltpu.sync_copy(pidx3.at[s], pidx)
    pltpu.sync_copy(didx3.at[s], didx)
    pltpu.sync_copy(init.at[pl.ds(s * dst_rows, dst_rows), pl.ds(c * HC, HC)],
                    acc.at[pl.ds(s * dst_rows, dst_rows)])
    plsc.subcore_barrier()

    def chunk_body(k, carry):
      hr = pltpu.async_copy(src3.at[c].at[pidx.at[k]], rows, sem_r)
      pltpu.sync_copy(widx3.at[s].at[k], woff)
      hw = pltpu.async_copy(wtab_hbm.at[woff], wv, sem_w)
      hr.wait()
      hw.wait()
      for g in range(CHUNK // LANES):
        sl = pl.ds(g * LANES, LANES)
        s16 = wv[sl]
        for i in range(LANES):
          r = g * LANES + i
          spl = jnp.full((LANES,), s16[i])
          for q in range(HC // LANES):
            qs = pl.ds(q * LANES, LANES)
            rows[r, qs] = rows[r, qs] * spl
      pltpu.sync_copy(rows, acc.at[didx.at[k]], add=True)
      return carry

    lax.fori_loop(0, NCHUNKS, chunk_body, 0)

    plsc.subcore_barrier()
    pltpu.sync_copy(acc.at[pl.ds(s * dst_rows, dst_rows)],
                    out.at[pl.ds(s * dst_rows, dst_rows), pl.ds(c * HC, HC)])

  return phase


_sc_v2e = _sc_phase(NV, NE)
_sc_e2v = _sc_phase(NE, NV)


BN_V = 2000
BN_E = 2000


def _tc1_body(v_ref, vw_ref, W_ref, b_ref, vew_ref, vpre_ref):
  x = v_ref[...]
  act = jnp.maximum(
      jnp.dot(x, W_ref[...], preferred_element_type=jnp.float32) + b_ref[...],
      0.0)
  vw = vw_ref[...]
  aw = act * vw
  vew_ref[0] = aw[:, :HC]
  vew_ref[1] = aw[:, HC:]
  vpre_ref[...] = x * vw


def _tc2_body(acc_ref, ew_ref, ers_ref, W_ref, b_ref, eout_ref, evw_ref):
  e_out = acc_ref[...] * (1.0 / ers_ref[...])
  act = jnp.maximum(
      jnp.dot(e_out, W_ref[...], preferred_element_type=jnp.float32)
      + b_ref[...], 0.0)
  eout_ref[...] = e_out
  aw = act * ew_ref[...]
  evw_ref[0] = aw[:, :HC]
  evw_ref[1] = aw[:, HC:]


def _tc3_body(acc_ref, vrs_ref, vout_ref):
  vout_ref[...] = acc_ref[...] * (1.0 / vrs_ref[...])


def _row_spec(bn, width):
  return pl.BlockSpec((bn, width), lambda i: (i, 0))


def _half_spec(bn):
  return pl.BlockSpec((2, bn, HC), lambda i: (0, i, 0))


def _full_spec(shape):
  return pl.BlockSpec(shape, lambda i: (0, 0))


_tc1 = pl.pallas_call(
    _tc1_body,
    grid=(NV // BN_V,),
    in_specs=[_row_spec(BN_V, NH), _row_spec(BN_V, 1),
              _full_spec((NH, NH)), _full_spec((1, NH))],
    out_specs=[_half_spec(BN_V), _row_spec(BN_V, NH)],
    out_shape=[jax.ShapeDtypeStruct((2, NV, HC), jnp.float32),
               jax.ShapeDtypeStruct((NV, NH), jnp.float32)],
)

_tc2 = pl.pallas_call(
    _tc2_body,
    grid=(NE // BN_E,),
    in_specs=[_row_spec(BN_E, NH), _row_spec(BN_E, 1), _row_spec(BN_E, 1),
              _full_spec((NH, NH)), _full_spec((1, NH))],
    out_specs=[_row_spec(BN_E, NH), _half_spec(BN_E)],
    out_shape=[jax.ShapeDtypeStruct((NE, NH), jnp.float32),
               jax.ShapeDtypeStruct((2, NE, HC), jnp.float32)],
)

_tc3 = pl.pallas_call(
    _tc3_body,
    grid=(NV // BN_V,),
    in_specs=[_row_spec(BN_V, NH), _row_spec(BN_V, 1)],
    out_specs=_row_spec(BN_V, NH),
    out_shape=jax.ShapeDtypeStruct((NV, NH), jnp.float32),
)


@jax.jit
def kernel(v, e, player_idx, game_idx, idx, W_v2e, W_e2v, b_v, b_e,
           v_weight, e_weight, v_reg_weight, e_reg_weight,
           v_reg_sum, e_reg_sum):
  pidx3 = player_idx.reshape(NTILES, NCHUNKS, CHUNK)
  gidx3 = game_idx.reshape(NTILES, NCHUNKS, CHUNK)
  widx3 = idx.reshape(NTILES, NCHUNKS, CHUNK)

  ve_w3, v_pre = _tc1(v, v_weight, W_v2e, b_v.reshape(1, NH))

  acc_e = _sc_v2e(ve_w3, e, pidx3, gidx3, widx3, v_reg_weight.reshape(NI))

  e_out, evw3 = _tc2(acc_e, e_weight, e_reg_sum, W_e2v, b_e.reshape(1, NH))

  acc_v = _sc_e2v(evw3, v_pre, gidx3, pidx3, widx3, e_reg_weight.reshape(NI))

  v_out = _tc3(acc_v, v_reg_sum)

  return (v_out, e_out)


# speedup vs baseline: 4.3333x; 1.1886x over previous
import functools

import jax
import jax.numpy as jnp
from jax import lax
from jax.experimental import pallas as pl
from jax.experimental.pallas import tpu as pltpu
from jax.experimental.pallas import tpu_sc as plsc

NV = 10000
NE = 20000
NH = 128
NI = 320000

HC = NH // 2
NCORES = 2
NTILES = 16
LANES = 16
CHUNK = 80
PER_TILE = NI // NTILES
NCHUNKS = PER_TILE // CHUNK


def _sc_phase(n_src, n_dst):
  dst_rows = n_dst // NTILES

  mesh = plsc.VectorSubcoreMesh(
      core_axis_name="c", subcore_axis_name="s",
      num_cores=NCORES, num_subcores=NTILES)

  @functools.partial(
      pl.kernel,
      out_type=jax.ShapeDtypeStruct((n_dst, NH), jnp.float32),
      mesh=mesh,
      compiler_params=pltpu.CompilerParams(use_tc_tiling_on_sc=False,
                                           needs_layout_passes=False),
      scratch_types=dict(
          acc=pltpu.VMEM_SHARED((n_dst, HC), jnp.float32),
          pidx=pltpu.VMEM((NCHUNKS, CHUNK), jnp.int32),
          didx=pltpu.VMEM((NCHUNKS, CHUNK), jnp.int32),
          woff=pltpu.VMEM((2, CHUNK), jnp.int32),
          rows=pltpu.VMEM((CHUNK, HC), jnp.float32),
          wv=pltpu.VMEM((CHUNK,), jnp.float32),
          sem_r=pltpu.SemaphoreType.DMA,
          sem_w=pltpu.SemaphoreType.DMA,
      ),
  )
  def phase(src3, init, pidx3, didx3, widx3, wtab_hbm, out,
            acc, pidx, didx, woff, rows, wv, sem_r, sem_w):
    c = lax.axis_index("c")
    s = lax.axis_index("s")

    pltpu.sync_copy(pidx3.at[s], pidx)
    pltpu.sync_copy(didx3.at[s], didx)
    pltpu.sync_copy(init.at[pl.ds(s * dst_rows, dst_rows), pl.ds(c * HC, HC)],
                    acc.at[pl.ds(s * dst_rows, dst_rows)])
    pltpu.sync_copy(widx3.at[s].at[0], woff.at[0])
    plsc.subcore_barrier()

    def chunk_body(k, carry):
      cur = lax.rem(k, 2)
      hr = pltpu.async_copy(src3.at[c].at[pidx.at[k]], rows, sem_r)
      hw = pltpu.async_copy(wtab_hbm.at[woff.at[cur]], wv, sem_w)
      kn = jnp.minimum(k + 1, NCHUNKS - 1)
      pltpu.sync_copy(widx3.at[s].at[kn], woff.at[1 - cur])
      hr.wait()
      hw.wait()
      for g in range(CHUNK // LANES):
        sl = pl.ds(g * LANES, LANES)
        s16 = wv[sl]
        for i in range(LANES):
          r = g * LANES + i
          spl = jnp.full((LANES,), s16[i])
          for q in range(HC // LANES):
            qs = pl.ds(q * LANES, LANES)
            rows[r, qs] = rows[r, qs] * spl
      pltpu.sync_copy(rows, acc.at[didx.at[k]], add=True)
      return carry

    lax.fori_loop(0, NCHUNKS, chunk_body, 0)

    plsc.subcore_barrier()
    pltpu.sync_copy(acc.at[pl.ds(s * dst_rows, dst_rows)],
                    out.at[pl.ds(s * dst_rows, dst_rows), pl.ds(c * HC, HC)])

  return phase


_sc_v2e = _sc_phase(NV, NE)
_sc_e2v = _sc_phase(NE, NV)


BN_V = 2000
BN_E = 2000


def _tc1_body(v_ref, vw_ref, W_ref, b_ref, vew_ref, vpre_ref):
  x = v_ref[...]
  act = jnp.maximum(
      jnp.dot(x, W_ref[...], preferred_element_type=jnp.float32) + b_ref[...],
      0.0)
  vw = vw_ref[...]
  aw = act * vw
  vew_ref[0] = aw[:, :HC]
  vew_ref[1] = aw[:, HC:]
  vpre_ref[...] = x * vw


def _tc2_body(acc_ref, ew_ref, ers_ref, W_ref, b_ref, eout_ref, evw_ref):
  e_out = acc_ref[...] * (1.0 / ers_ref[...])
  act = jnp.maximum(
      jnp.dot(e_out, W_ref[...], preferred_element_type=jnp.float32)
      + b_ref[...], 0.0)
  eout_ref[...] = e_out
  aw = act * ew_ref[...]
  evw_ref[0] = aw[:, :HC]
  evw_ref[1] = aw[:, HC:]


def _tc3_body(acc_ref, vrs_ref, vout_ref):
  vout_ref[...] = acc_ref[...] * (1.0 / vrs_ref[...])


def _row_spec(bn, width):
  return pl.BlockSpec((bn, width), lambda i: (i, 0))


def _half_spec(bn):
  return pl.BlockSpec((2, bn, HC), lambda i: (0, i, 0))


def _full_spec(shape):
  return pl.BlockSpec(shape, lambda i: (0, 0))


_tc1 = pl.pallas_call(
    _tc1_body,
    grid=(NV // BN_V,),
    in_specs=[_row_spec(BN_V, NH), _row_spec(BN_V, 1),
              _full_spec((NH, NH)), _full_spec((1, NH))],
    out_specs=[_half_spec(BN_V), _row_spec(BN_V, NH)],
    out_shape=[jax.ShapeDtypeStruct((2, NV, HC), jnp.float32),
               jax.ShapeDtypeStruct((NV, NH), jnp.float32)],
)

_tc2 = pl.pallas_call(
    _tc2_body,
    grid=(NE // BN_E,),
    in_specs=[_row_spec(BN_E, NH), _row_spec(BN_E, 1), _row_spec(BN_E, 1),
              _full_spec((NH, NH)), _full_spec((1, NH))],
    out_specs=[_row_spec(BN_E, NH), _half_spec(BN_E)],
    out_shape=[jax.ShapeDtypeStruct((NE, NH), jnp.float32),
               jax.ShapeDtypeStruct((2, NE, HC), jnp.float32)],
)

_tc3 = pl.pallas_call(
    _tc3_body,
    grid=(NV // BN_V,),
    in_specs=[_row_spec(BN_V, NH), _row_spec(BN_V, 1)],
    out_specs=_row_spec(BN_V, NH),
    out_shape=jax.ShapeDtypeStruct((NV, NH), jnp.float32),
)


@jax.jit
def kernel(v, e, player_idx, game_idx, idx, W_v2e, W_e2v, b_v, b_e,
           v_weight, e_weight, v_reg_weight, e_reg_weight,
           v_reg_sum, e_reg_sum):
  pidx3 = player_idx.reshape(NTILES, NCHUNKS, CHUNK)
  gidx3 = game_idx.reshape(NTILES, NCHUNKS, CHUNK)
  widx3 = idx.reshape(NTILES, NCHUNKS, CHUNK)

  ve_w3, v_pre = _tc1(v, v_weight, W_v2e, b_v.reshape(1, NH))

  acc_e = _sc_v2e(ve_w3, e, pidx3, gidx3, widx3, v_reg_weight.reshape(NI))

  e_out, evw3 = _tc2(acc_e, e_weight, e_reg_sum, W_e2v, b_e.reshape(1, NH))

  acc_v = _sc_e2v(evw3, v_pre, gidx3, pidx3, widx3, e_reg_weight.reshape(NI))

  v_out = _tc3(acc_v, v_reg_sum)

  return (v_out, e_out)
